# baseline (speedup 1.0000x reference)
import functools

import jax
import jax.numpy as jnp
from jax import lax
from jax.experimental import pallas as pl
from jax.experimental.pallas import tpu as pltpu
from jax.experimental.pallas import tpu_sc as plsc

NUM_ROWS = 1000000
BATCH = 16384
EMBED_DIM = 32
SUBLANE = 8
DTILE = EMBED_DIM // SUBLANE
NUM_CORES = 2
NUM_SUBCORES = 16
NUM_WORKERS = NUM_CORES * NUM_SUBCORES
BPW = BATCH // NUM_WORKERS
LPP = 32
NPASS = BPW // LPP
SLAB = LPP * 16


@functools.partial(
    pl.kernel,
    mesh=plsc.VectorSubcoreMesh(core_axis_name="c", subcore_axis_name="s"),
    compiler_params=pltpu.CompilerParams(needs_layout_passes=False),
    out_type=jax.ShapeDtypeStruct((BATCH,), jnp.float32),
    scratch_types=[
        pltpu.VMEM((BPW + 16,), jnp.int32),
        pltpu.VMEM((BPW + 16,), jnp.int32),
        pltpu.VMEM((DTILE, SUBLANE, SLAB), jnp.float32),
        pltpu.VMEM((DTILE, SUBLANE, SLAB), jnp.float32),
        pltpu.VMEM((DTILE, SUBLANE, SLAB), jnp.float32),
        pltpu.VMEM((DTILE, SUBLANE, SLAB), jnp.float32),
        pltpu.VMEM((BPW,), jnp.float32),
        pltpu.SemaphoreType.DMA,
        pltpu.SemaphoreType.DMA,
        pltpu.SemaphoreType.DMA,
        pltpu.SemaphoreType.DMA,
    ],
)
def _mf_score_sc(uidx_hbm, iidx_hbm, utab_hbm, itab_hbm, out_hbm,
                 uidx_v, iidx_v, uvalA, ivalA, uvalB, ivalB, out_v,
                 usemA, isemA, usemB, isemB):
    wid = lax.axis_index("s") * NUM_CORES + lax.axis_index("c")
    base = wid * BPW

    pltpu.sync_copy(uidx_hbm.at[pl.ds(base, BPW)], uidx_v.at[pl.ds(0, BPW)])
    pltpu.sync_copy(iidx_hbm.at[pl.ds(base, BPW)], iidx_v.at[pl.ds(0, BPW)])
    uidx_v[pl.ds(BPW, 16)] = jnp.zeros((16,), jnp.int32)
    iidx_v[pl.ds(BPW, 16)] = jnp.zeros((16,), jnp.int32)

    def fire(p, uslab, islab, usem, isem):
        @plsc.parallel_loop(0, LPP, unroll=4)
        def fk(k):
            j = p * LPP + k
            uv = uidx_v[pl.ds(j, 16)]
            iv = iidx_v[pl.ds(j, 16)]
            ub = pl.multiple_of((uv[0] >> 4) << 4, 16)
            ib = pl.multiple_of((iv[0] >> 4) << 4, 16)
            pltpu.async_copy(utab_hbm.at[:, :, pl.ds(ub, 16)],
                             uslab.at[:, :, pl.ds(k * 16, 16)], usem)
            pltpu.async_copy(itab_hbm.at[:, :, pl.ds(ib, 16)],
                             islab.at[:, :, pl.ds(k * 16, 16)], isem)

    def drain(uslab, islab, usem, isem):
        pltpu.make_async_copy(utab_hbm.at[:, :, pl.ds(0, SLAB)],
                              uslab, usem).wait()
        pltpu.make_async_copy(itab_hbm.at[:, :, pl.ds(0, SLAB)],
                              islab, isem).wait()

    def compute(p, uslab, islab):
        lane_base = lax.iota(jnp.int32, 16) * 16
        for g in range(LPP // 16):
            u16 = uidx_v[pl.ds(p * LPP + g * 16, 16)]
            i16 = iidx_v[pl.ds(p * LPP + g * 16, 16)]
            ulanes = lane_base + g * 256 + (u16 & 15)
            ilanes = lane_base + g * 256 + (i16 & 15)
            acc = jnp.zeros((16,), jnp.float32)
            for t in range(DTILE):
                tt = jnp.full((16,), t, jnp.int32)
                for s in range(SUBLANE):
                    ss = jnp.full((16,), s, jnp.int32)
                    u = plsc.load_gather(uslab, [tt, ss, ulanes])
                    v = plsc.load_gather(islab, [tt, ss, ilanes])
                    acc = acc + u * v
            out_v[pl.ds(p * LPP + g * 16, 16)] = acc

    fire(0, uvalA, ivalA, usemA, isemA)

    def body(h, carry):
        p = h * 2
        fire(p + 1, uvalB, ivalB, usemB, isemB)
        drain(uvalA, ivalA, usemA, isemA)
        compute(p, uvalA, ivalA)
        fire(p + 2, uvalA, ivalA, usemA, isemA)
        drain(uvalB, ivalB, usemB, isemB)
        compute(p + 1, uvalB, ivalB)
        return carry

    lax.fori_loop(0, NPASS // 2 - 1, body, 0)

    fire(NPASS - 1, uvalB, ivalB, usemB, isemB)
    drain(uvalA, ivalA, usemA, isemA)
    compute(NPASS - 2, uvalA, ivalA)
    drain(uvalB, ivalB, usemB, isemB)
    compute(NPASS - 1, uvalB, ivalB)

    pltpu.sync_copy(out_v, out_hbm.at[pl.ds(base, BPW)])


def kernel(user_idx, item_idx, user_table, item_table):
    ut3 = user_table.T.reshape(DTILE, SUBLANE, NUM_ROWS)
    it3 = item_table.T.reshape(DTILE, SUBLANE, NUM_ROWS)
    return _mf_score_sc(user_idx.astype(jnp.int32),
                        item_idx.astype(jnp.int32), ut3, it3)


# speedup vs baseline: 1.0706x; 1.0706x over previous
import functools

import jax
import jax.numpy as jnp
from jax import lax
from jax.experimental import pallas as pl
from jax.experimental.pallas import tpu as pltpu
from jax.experimental.pallas import tpu_sc as plsc

NUM_ROWS = 1000000
BATCH = 16384
EMBED_DIM = 32
SUBLANE = 8
DTILE = EMBED_DIM // SUBLANE
NUM_CORES = 2
NUM_SUBCORES = 16
NUM_WORKERS = NUM_CORES * NUM_SUBCORES
BPW = BATCH // NUM_WORKERS
LPP = 16
NPASS = BPW // LPP
SLAB = LPP * 16


@functools.partial(
    pl.kernel,
    mesh=plsc.VectorSubcoreMesh(core_axis_name="c", subcore_axis_name="s"),
    compiler_params=pltpu.CompilerParams(needs_layout_passes=False),
    out_type=jax.ShapeDtypeStruct((BATCH,), jnp.float32),
    scratch_types=[
        pltpu.VMEM((BPW + 16,), jnp.int32),
        pltpu.VMEM((BPW + 16,), jnp.int32),
        pltpu.VMEM((DTILE, SUBLANE, SLAB), jnp.float32),
        pltpu.VMEM((DTILE, SUBLANE, SLAB), jnp.float32),
        pltpu.VMEM((DTILE, SUBLANE, SLAB), jnp.float32),
        pltpu.VMEM((DTILE, SUBLANE, SLAB), jnp.float32),
        pltpu.VMEM((BPW,), jnp.float32),
        pltpu.SemaphoreType.DMA,
        pltpu.SemaphoreType.DMA,
        pltpu.SemaphoreType.DMA,
        pltpu.SemaphoreType.DMA,
    ],
)
def _mf_score_sc(uidx_hbm, iidx_hbm, utab_hbm, itab_hbm, out_hbm,
                 uidx_v, iidx_v, uvalA, ivalA, uvalB, ivalB, out_v,
                 usemA, isemA, usemB, isemB):
    wid = lax.axis_index("s") * NUM_CORES + lax.axis_index("c")
    base = wid * BPW

    pltpu.sync_copy(uidx_hbm.at[pl.ds(base, BPW)], uidx_v.at[pl.ds(0, BPW)])
    pltpu.sync_copy(iidx_hbm.at[pl.ds(base, BPW)], iidx_v.at[pl.ds(0, BPW)])
    uidx_v[pl.ds(BPW, 16)] = jnp.zeros((16,), jnp.int32)
    iidx_v[pl.ds(BPW, 16)] = jnp.zeros((16,), jnp.int32)

    def fire(p, uslab, islab, usem, isem):
        def fk(k, carry):
            j = p * LPP + k
            uv = uidx_v[pl.ds(j, 16)]
            iv = iidx_v[pl.ds(j, 16)]
            ub = pl.multiple_of((uv[0] >> 4) << 4, 16)
            ib = pl.multiple_of((iv[0] >> 4) << 4, 16)
            pltpu.async_copy(utab_hbm.at[:, :, pl.ds(ub, 16)],
                             uslab.at[:, :, pl.ds(k * 16, 16)], usem)
            pltpu.async_copy(itab_hbm.at[:, :, pl.ds(ib, 16)],
                             islab.at[:, :, pl.ds(k * 16, 16)], isem)
            return carry
        lax.fori_loop(0, LPP, fk, 0, unroll=4)

    def drain(uslab, islab, usem, isem):
        pltpu.make_async_copy(utab_hbm.at[:, :, pl.ds(0, SLAB)],
                              uslab, usem).wait()
        pltpu.make_async_copy(itab_hbm.at[:, :, pl.ds(0, SLAB)],
                              islab, isem).wait()

    def compute(p, uslab, islab):
        lane_base = lax.iota(jnp.int32, 16) * 16
        for g in range(LPP // 16):
            u16 = uidx_v[pl.ds(p * LPP + g * 16, 16)]
            i16 = iidx_v[pl.ds(p * LPP + g * 16, 16)]
            ulanes = lane_base + g * 256 + (u16 & 15)
            ilanes = lane_base + g * 256 + (i16 & 15)
            acc = jnp.zeros((16,), jnp.float32)
            for t in range(DTILE):
                tt = jnp.full((16,), t, jnp.int32)
                for s in range(SUBLANE):
                    ss = jnp.full((16,), s, jnp.int32)
                    u = plsc.load_gather(uslab, [tt, ss, ulanes])
                    v = plsc.load_gather(islab, [tt, ss, ilanes])
                    acc = acc + u * v
            out_v[pl.ds(p * LPP + g * 16, 16)] = acc

    fire(0, uvalA, ivalA, usemA, isemA)

    def body(h, carry):
        p = h * 2
        fire(p + 1, uvalB, ivalB, usemB, isemB)
        drain(uvalA, ivalA, usemA, isemA)
        compute(p, uvalA, ivalA)
        fire(p + 2, uvalA, ivalA, usemA, isemA)
        drain(uvalB, ivalB, usemB, isemB)
        compute(p + 1, uvalB, ivalB)
        return carry

    lax.fori_loop(0, NPASS // 2 - 1, body, 0)

    fire(NPASS - 1, uvalB, ivalB, usemB, isemB)
    drain(uvalA, ivalA, usemA, isemA)
    compute(NPASS - 2, uvalA, ivalA)
    drain(uvalB, ivalB, usemB, isemB)
    compute(NPASS - 1, uvalB, ivalB)

    pltpu.sync_copy(out_v, out_hbm.at[pl.ds(base, BPW)])


def kernel(user_idx, item_idx, user_table, item_table):
    ut3 = user_table.T.reshape(DTILE, SUBLANE, NUM_ROWS)
    it3 = item_table.T.reshape(DTILE, SUBLANE, NUM_ROWS)
    return _mf_score_sc(user_idx.astype(jnp.int32),
                        item_idx.astype(jnp.int32), ut3, it3)


# speedup vs baseline: 1.1649x; 1.0881x over previous
import functools

import jax
import jax.numpy as jnp
from jax import lax
from jax.experimental import pallas as pl
from jax.experimental.pallas import tpu as pltpu
from jax.experimental.pallas import tpu_sc as plsc

NUM_ROWS = 1000000
BATCH = 16384
EMBED_DIM = 32
SUBLANE = 8
DTILE = EMBED_DIM // SUBLANE
NUM_CORES = 2
NUM_SUBCORES = 16
NUM_WORKERS = NUM_CORES * NUM_SUBCORES
BPW = BATCH // NUM_WORKERS
LPP = 16
NPASS = BPW // LPP
SLAB = LPP * 16


@functools.partial(
    pl.kernel,
    mesh=plsc.VectorSubcoreMesh(core_axis_name="c", subcore_axis_name="s"),
    compiler_params=pltpu.CompilerParams(needs_layout_passes=False),
    out_type=jax.ShapeDtypeStruct((BATCH,), jnp.float32),
    scratch_types=[
        pltpu.VMEM((BPW + 16,), jnp.int32),
        pltpu.VMEM((BPW + 16,), jnp.int32),
        pltpu.VMEM((DTILE, SUBLANE, SLAB), jnp.float32),
        pltpu.VMEM((DTILE, SUBLANE, SLAB), jnp.float32),
        pltpu.VMEM((DTILE, SUBLANE, SLAB), jnp.float32),
        pltpu.VMEM((DTILE, SUBLANE, SLAB), jnp.float32),
        pltpu.VMEM((BPW,), jnp.float32),
        pltpu.SemaphoreType.DMA,
        pltpu.SemaphoreType.DMA,
        pltpu.SemaphoreType.DMA,
        pltpu.SemaphoreType.DMA,
    ],
)
def _mf_score_sc(uidx_hbm, iidx_hbm, utab_hbm, itab_hbm, out_hbm,
                 uidx_v, iidx_v, uvalA, ivalA, uvalB, ivalB, out_v,
                 usemA, isemA, usemB, isemB):
    wid = lax.axis_index("s") * NUM_CORES + lax.axis_index("c")
    base = wid * BPW

    pltpu.sync_copy(uidx_hbm.at[pl.ds(base, BPW)], uidx_v.at[pl.ds(0, BPW)])
    pltpu.sync_copy(iidx_hbm.at[pl.ds(base, BPW)], iidx_v.at[pl.ds(0, BPW)])
    uidx_v[pl.ds(BPW, 16)] = jnp.zeros((16,), jnp.int32)
    iidx_v[pl.ds(BPW, 16)] = jnp.zeros((16,), jnp.int32)

    def fire(p, uslab, islab, usem, isem):
        def fk(k, carry):
            j = p * LPP + k
            uv = uidx_v[pl.ds(j, 16)]
            iv = iidx_v[pl.ds(j, 16)]
            ub = pl.multiple_of(uv[0] & -16, 16)
            ib = pl.multiple_of(iv[0] & -16, 16)
            pltpu.async_copy(utab_hbm.at[:, :, pl.ds(ub, 16)],
                             uslab.at[:, :, pl.ds(k * 16, 16)], usem)
            pltpu.async_copy(itab_hbm.at[:, :, pl.ds(ib, 16)],
                             islab.at[:, :, pl.ds(k * 16, 16)], isem)
            return carry
        lax.fori_loop(0, LPP, fk, 0)

    def drain(uslab, islab, usem, isem):
        pltpu.make_async_copy(utab_hbm.at[:, :, pl.ds(0, SLAB)],
                              uslab, usem).wait()
        pltpu.make_async_copy(itab_hbm.at[:, :, pl.ds(0, SLAB)],
                              islab, isem).wait()

    def compute(p, uslab, islab):
        lane_base = lax.iota(jnp.int32, 16) * 16
        for g in range(LPP // 16):
            u16 = uidx_v[pl.ds(p * LPP + g * 16, 16)]
            i16 = iidx_v[pl.ds(p * LPP + g * 16, 16)]
            ulanes = lane_base + g * 256 + (u16 & 15)
            ilanes = lane_base + g * 256 + (i16 & 15)
            acc = jnp.zeros((16,), jnp.float32)
            for t in range(DTILE):
                tt = jnp.full((16,), t, jnp.int32)
                for s in range(SUBLANE):
                    ss = jnp.full((16,), s, jnp.int32)
                    u = plsc.load_gather(uslab, [tt, ss, ulanes])
                    v = plsc.load_gather(islab, [tt, ss, ilanes])
                    acc = acc + u * v
            out_v[pl.ds(p * LPP + g * 16, 16)] = acc

    fire(0, uvalA, ivalA, usemA, isemA)

    def body(h, carry):
        p = h * 2
        fire(p + 1, uvalB, ivalB, usemB, isemB)
        drain(uvalA, ivalA, usemA, isemA)
        compute(p, uvalA, ivalA)
        fire(p + 2, uvalA, ivalA, usemA, isemA)
        drain(uvalB, ivalB, usemB, isemB)
        compute(p + 1, uvalB, ivalB)
        return carry

    lax.fori_loop(0, NPASS // 2 - 1, body, 0)

    fire(NPASS - 1, uvalB, ivalB, usemB, isemB)
    drain(uvalA, ivalA, usemA, isemA)
    compute(NPASS - 2, uvalA, ivalA)
    drain(uvalB, ivalB, usemB, isemB)
    compute(NPASS - 1, uvalB, ivalB)

    pltpu.sync_copy(out_v, out_hbm.at[pl.ds(base, BPW)])


def kernel(user_idx, item_idx, user_table, item_table):
    ut3 = user_table.T.reshape(DTILE, SUBLANE, NUM_ROWS)
    it3 = item_table.T.reshape(DTILE, SUBLANE, NUM_ROWS)
    return _mf_score_sc(user_idx.astype(jnp.int32),
                        item_idx.astype(jnp.int32), ut3, it3)
